# Initial kernel scaffold; baseline (speedup 1.0000x reference)
import jax
import jax.numpy as jnp
from jax.experimental import pallas as pl


def kernel(input_tensor, token_emb, segment_emb, positional_emb):
    raise NotImplementedError("write your pallas kernel here")


# speedup vs baseline: 6.4358x; 6.4358x over previous
import functools

import jax
import jax.numpy as jnp
from jax import lax
from jax.experimental import pallas as pl
from jax.experimental.pallas import tpu as pltpu
from jax.experimental.pallas import tpu_sc as plsc

NC = 2
NS = 16
L = 16
D = 128
C = 128


@functools.partial(jax.jit, static_argnums=(0, 1))
def _sc_joint_embedding(seq, n_flat, idx, tok, seg, pos):
    nw = NC * NS
    rows_per_w = n_flat // nw
    n_chunks = rows_per_w // C
    mesh = plsc.VectorSubcoreMesh(core_axis_name="c", subcore_axis_name="s")

    @functools.partial(
        pl.kernel,
        out_type=jax.ShapeDtypeStruct((n_flat, D), jnp.float32),
        mesh=mesh,
        scratch_types=[
            pltpu.VMEM((2 * seq, D), jnp.float32),
            pltpu.VMEM((2, D), jnp.float32),
            pltpu.VMEM((C,), jnp.int32),
            pltpu.VMEM((C, D), jnp.float32),
            pltpu.SemaphoreType.DMA,
        ],
    )
    def k(idx_hbm, tok_hbm, seg_hbm, pos_hbm, out_hbm,
          bias2, segv, idxv, rows, sem):
        wid = lax.axis_index("s") * NC + lax.axis_index("c")

        pltpu.sync_copy(pos_hbm.at[pl.ds(0, seq)], bias2.at[pl.ds(0, seq)])
        pltpu.sync_copy(seg_hbm.at[pl.ds(0, 2)], segv)
        half = seq // 2 + 1

        def add_seg_row(srow, lo, hi):
            def body(s, carry):
                for c in range(D // L):
                    plsc.addupdate(bias2.at[s, pl.ds(c * L, L)],
                                   segv[srow, pl.ds(c * L, L)])
                return carry
            lax.fori_loop(lo, hi, body, 0)

        add_seg_row(0, 0, half)
        add_seg_row(1, half, seq)

        def dup_row(s, carry):
            for c in range(D // L):
                bias2[s + seq, pl.ds(c * L, L)] = bias2[s, pl.ds(c * L, L)]
            return carry
        lax.fori_loop(0, seq, dup_row, 0)

        wbase = wid * rows_per_w

        def chunk(j, carry):
            base = wbase + j * C
            s0 = lax.rem(j * C, seq)
            pltpu.sync_copy(idx_hbm.at[pl.ds(base, C)], idxv)
            pltpu.async_copy(tok_hbm.at[idxv], rows, sem).wait()

            def add_bias(r, inner):
                br = s0 + r
                for c in range(D // L):
                    plsc.addupdate(rows.at[r, pl.ds(c * L, L)],
                                   bias2[br, pl.ds(c * L, L)])
                return inner
            lax.fori_loop(0, C, add_bias, 0)

            pltpu.sync_copy(rows, out_hbm.at[pl.ds(base, C)])
            return carry

        lax.fori_loop(0, n_chunks, chunk, 0)

    return k(idx, tok, seg, pos)


def kernel(input_tensor, token_emb, segment_emb, positional_emb):
    b, s = input_tensor.shape
    n_flat = b * s
    idx = input_tensor.reshape(n_flat)
    out = _sc_joint_embedding(s, n_flat, idx, token_emb,
                              segment_emb, positional_emb)
    return out.reshape(b, s, D)


# speedup vs baseline: 10.1014x; 1.5696x over previous
import functools

import jax
import jax.numpy as jnp
from jax import lax
from jax.experimental import pallas as pl
from jax.experimental.pallas import tpu as pltpu
from jax.experimental.pallas import tpu_sc as plsc

NC = 2
NS = 16
L = 16
D = 128
C = 128
NBUF = 4


@functools.partial(jax.jit, static_argnums=(0, 1))
def _sc_joint_embedding(seq, n_flat, idx, tok, seg, pos):
    nw = NC * NS
    rows_per_w = n_flat // nw
    n_chunks = rows_per_w // C
    s0max = max((j * C) % seq for j in range(n_chunks))
    bias_rows = s0max + C
    mesh = plsc.VectorSubcoreMesh(core_axis_name="c", subcore_axis_name="s")

    @functools.partial(
        pl.kernel,
        out_type=jax.ShapeDtypeStruct((n_flat, D), jnp.float32),
        mesh=mesh,
        scratch_types=[
            pltpu.VMEM((bias_rows, D), jnp.float32),
            pltpu.VMEM((2, D), jnp.float32),
            pltpu.VMEM((NBUF, C), jnp.int32),
            pltpu.VMEM((NBUF, C, D), jnp.float32),
            [pltpu.SemaphoreType.DMA] * NBUF,
            [pltpu.SemaphoreType.DMA] * NBUF,
            [pltpu.SemaphoreType.DMA] * NBUF,
        ],
    )
    def k(idx_hbm, tok_hbm, seg_hbm, pos_hbm, out_hbm,
          bias, segv, idxv, rows, isem, gsem, ssem):
        wid = lax.axis_index("s") * NC + lax.axis_index("c")
        wbase = wid * rows_per_w

        pltpu.sync_copy(pos_hbm.at[pl.ds(0, seq)], bias.at[pl.ds(0, seq)])
        pltpu.sync_copy(seg_hbm.at[pl.ds(0, 2)], segv)
        half = seq // 2 + 1

        def add_seg_row(srow, lo, hi):
            def body(s, carry):
                for c in range(D // L):
                    plsc.addupdate(bias.at[s, pl.ds(c * L, L)],
                                   segv[srow, pl.ds(c * L, L)])
                return carry
            lax.fori_loop(lo, hi, body, 0)

        add_seg_row(0, 0, half)
        add_seg_row(1, half, seq)

        def dup_row(s, carry):
            for c in range(D // L):
                bias[s + seq, pl.ds(c * L, L)] = bias[s, pl.ds(c * L, L)]
            return carry
        lax.fori_loop(0, bias_rows - seq, dup_row, 0)

        def idx_copy(t, b):
            return pltpu.make_async_copy(
                idx_hbm.at[pl.ds(wbase + t * C, C)], idxv.at[b], isem[b])

        def gat_copy(t, b):
            return pltpu.make_async_copy(
                tok_hbm.at[idxv.at[b]], rows.at[b], gsem[b])

        def scat_copy(t, b):
            return pltpu.make_async_copy(
                rows.at[b], out_hbm.at[pl.ds(wbase + t * C, C)], ssem[b])

        for t in range(3):
            idx_copy(t, t).start()
        for t in range(2):
            idx_copy(t, t).wait()
            gat_copy(t, t).start()

        def group(g, carry):
            for b in range(NBUF):
                j = g * NBUF + b
                t3 = j + 3
                b3 = (b + 3) % NBUF

                @pl.when(t3 < n_chunks)
                def _():
                    idx_copy(t3, b3).start()

                t2 = j + 2
                b2 = (b + 2) % NBUF

                @pl.when(t2 < n_chunks)
                def _():
                    @pl.when(j >= 2)
                    def _():
                        scat_copy(j - 2, b2).wait()
                    idx_copy(t2, b2).wait()
                    gat_copy(t2, b2).start()

                gat_copy(j, b).wait()
                s0 = lax.rem(j * C, seq)

                def add_bias(r, inner):
                    br = s0 + r
                    for c in range(D // L):
                        plsc.addupdate(rows.at[b, r, pl.ds(c * L, L)],
                                       bias[br, pl.ds(c * L, L)])
                    return inner
                lax.fori_loop(0, C, add_bias, 0)

                scat_copy(j, b).start()
            return carry

        lax.fori_loop(0, n_chunks // NBUF, group, 0)

        for b in range(NBUF):
            scat_copy(n_chunks - NBUF + b, b).wait()

    return k(idx, tok, seg, pos)


def kernel(input_tensor, token_emb, segment_emb, positional_emb):
    b, s = input_tensor.shape
    n_flat = b * s
    idx = input_tensor.reshape(n_flat)
    out = _sc_joint_embedding(s, n_flat, idx, token_emb,
                              segment_emb, positional_emb)
    return out.reshape(b, s, D)


# speedup vs baseline: 28.2984x; 2.8014x over previous
import functools

import jax
import jax.numpy as jnp
from jax import lax
from jax.experimental import pallas as pl
from jax.experimental.pallas import tpu as pltpu
from jax.experimental.pallas import tpu_sc as plsc

NC = 2
NS = 16
L = 16
D = 128
C = 128
NBUF = 4
RU = 8


@functools.partial(jax.jit, static_argnums=(0, 1))
def _sc_joint_embedding(batch, seq, idx_t, tok, seg, pos):
    nw = NC * NS
    n_chunks = seq
    half = seq // 2
    mesh = plsc.VectorSubcoreMesh(core_axis_name="c", subcore_axis_name="s")

    @functools.partial(
        pl.kernel,
        out_type=jax.ShapeDtypeStruct((batch, seq, D), jnp.float32),
        mesh=mesh,
        scratch_types=[
            pltpu.VMEM((seq, D), jnp.float32),
            pltpu.VMEM((2, D), jnp.float32),
            pltpu.VMEM((NBUF, C), jnp.int32),
            pltpu.VMEM((NBUF, C, D), jnp.float32),
            [pltpu.SemaphoreType.DMA] * NBUF,
            [pltpu.SemaphoreType.DMA] * NBUF,
            [pltpu.SemaphoreType.DMA] * NBUF,
        ],
    )
    def k(idx_hbm, tok_hbm, seg_hbm, pos_hbm, out_hbm,
          posb, segv, idxv, rows, isem, gsem, ssem):
        wid = lax.axis_index("s") * NC + lax.axis_index("c")
        wb = wid * C

        pltpu.sync_copy(pos_hbm.at[pl.ds(0, seq)], posb)
        pltpu.sync_copy(seg_hbm.at[pl.ds(0, 2)], segv)

        def idx_copy(t, b):
            return pltpu.make_async_copy(
                idx_hbm.at[pl.ds(t * batch + wb, C)], idxv.at[b], isem[b])

        def gat_copy(t, b):
            return pltpu.make_async_copy(
                tok_hbm.at[idxv.at[b]], rows.at[b], gsem[b])

        def scat_copy(t, b):
            return pltpu.make_async_copy(
                rows.at[b], out_hbm.at[pl.ds(wb, C), t], ssem[b])

        for t in range(3):
            idx_copy(t, t).start()
        for t in range(2):
            idx_copy(t, t).wait()
            gat_copy(t, t).start()

        def group(g, carry):
            for b in range(NBUF):
                j = g * NBUF + b
                t3 = j + 3
                b3 = (b + 3) % NBUF

                @pl.when(t3 < n_chunks)
                def _():
                    idx_copy(t3, b3).start()

                t2 = j + 2
                b2 = (b + 2) % NBUF

                @pl.when(t2 < n_chunks)
                def _():
                    @pl.when(j >= 2)
                    def _():
                        scat_copy(j - 2, b2).wait()
                    idx_copy(t2, b2).wait()
                    gat_copy(t2, b2).start()

                gat_copy(j, b).wait()

                srow = jnp.where(j > half, 1, 0)
                bias_c = [posb[j, pl.ds(c * L, L)] + segv[srow, pl.ds(c * L, L)]
                          for c in range(D // L)]

                def add_bias(r, inner):
                    base = r * RU
                    for k in range(RU):
                        for c in range(D // L):
                            plsc.addupdate(
                                rows.at[b, base + k, pl.ds(c * L, L)],
                                bias_c[c])
                    return inner
                lax.fori_loop(0, C // RU, add_bias, 0)

                scat_copy(j, b).start()
            return carry

        lax.fori_loop(0, n_chunks // NBUF, group, 0)

        for b in range(NBUF):
            scat_copy(n_chunks - NBUF + b, b).wait()

    return k(idx_t, tok, seg, pos)


def kernel(input_tensor, token_emb, segment_emb, positional_emb):
    b, s = input_tensor.shape
    idx_t = input_tensor.T.reshape(b * s)
    return _sc_joint_embedding(b, s, idx_t, token_emb,
                               segment_emb, positional_emb)


# speedup vs baseline: 28.3658x; 1.0024x over previous
import functools

import jax
import jax.numpy as jnp
from jax import lax
from jax.experimental import pallas as pl
from jax.experimental.pallas import tpu as pltpu
from jax.experimental.pallas import tpu_sc as plsc

NC = 2
NS = 16
L = 16
D = 128
C = 128
NBUF = 5
RU = 8


@functools.partial(jax.jit, static_argnums=(0, 1))
def _sc_joint_embedding(batch, seq, idx_t, tok, seg, pos):
    nw = NC * NS
    n_chunks = seq
    half = seq // 2
    mesh = plsc.VectorSubcoreMesh(core_axis_name="c", subcore_axis_name="s")

    @functools.partial(
        pl.kernel,
        out_type=jax.ShapeDtypeStruct((batch, seq, D), jnp.float32),
        mesh=mesh,
        scratch_types=[
            pltpu.VMEM((seq, D), jnp.float32),
            pltpu.VMEM((2, D), jnp.float32),
            pltpu.VMEM((NBUF, C), jnp.int32),
            pltpu.VMEM((NBUF, C, D), jnp.float32),
            [pltpu.SemaphoreType.DMA] * NBUF,
            [pltpu.SemaphoreType.DMA] * NBUF,
            [pltpu.SemaphoreType.DMA] * NBUF,
        ],
    )
    def k(idx_hbm, tok_hbm, seg_hbm, pos_hbm, out_hbm,
          posb, segv, idxv, rows, isem, gsem, ssem):
        wid = lax.axis_index("s") * NC + lax.axis_index("c")
        wb = wid * C

        pltpu.sync_copy(pos_hbm.at[pl.ds(0, seq)], posb)
        pltpu.sync_copy(seg_hbm.at[pl.ds(0, 2)], segv)

        def idx_copy(t, b):
            return pltpu.make_async_copy(
                idx_hbm.at[pl.ds(t * batch + wb, C)], idxv.at[b], isem[b])

        def gat_copy(t, b):
            return pltpu.make_async_copy(
                tok_hbm.at[idxv.at[b]], rows.at[b], gsem[b])

        def scat_copy(t, b):
            return pltpu.make_async_copy(
                rows.at[b], out_hbm.at[pl.ds(wb, C), t], ssem[b])

        for t in range(NBUF - 1):
            idx_copy(t, t).start()
        for t in range(NBUF - 2):
            idx_copy(t, t).wait()
            gat_copy(t, t).start()

        def group(g, carry):
            for b in range(NBUF):
                j = g * NBUF + b
                ti = j + NBUF - 1
                bi = (b + NBUF - 1) % NBUF

                @pl.when(ti < n_chunks)
                def _():
                    idx_copy(ti, bi).start()

                tg = j + NBUF - 2
                bg = (b + NBUF - 2) % NBUF

                @pl.when(tg < n_chunks)
                def _():
                    @pl.when(j >= 2)
                    def _():
                        scat_copy(j - 2, bg).wait()
                    idx_copy(tg, bg).wait()
                    gat_copy(tg, bg).start()

                gat_copy(j, b).wait()

                srow = jnp.where(j > half, 1, 0)
                bias_c = [posb[j, pl.ds(c * L, L)] + segv[srow, pl.ds(c * L, L)]
                          for c in range(D // L)]

                def add_bias(r, inner):
                    base = r * RU
                    for k in range(RU):
                        for c in range(D // L):
                            plsc.addupdate(
                                rows.at[b, base + k, pl.ds(c * L, L)],
                                bias_c[c])
                    return inner
                lax.fori_loop(0, C // RU, add_bias, 0)

                scat_copy(j, b).start()
            return carry

        lax.fori_loop(0, n_chunks // NBUF, group, 0)

        for b in range(NBUF):
            scat_copy(n_chunks - NBUF + b, b).wait()

    return k(idx_t, tok, seg, pos)


def kernel(input_tensor, token_emb, segment_emb, positional_emb):
    b, s = input_tensor.shape
    idx_t = input_tensor.T.reshape(b * s)
    return _sc_joint_embedding(b, s, idx_t, token_emb,
                               segment_emb, positional_emb)
